# baseline (speedup 1.0000x reference)
import functools

import jax
import jax.numpy as jnp
from jax import lax
from jax.experimental import pallas as pl
from jax.experimental.pallas import tpu as pltpu
from jax.experimental.pallas import tpu_sc as plsc

_GLOBAL_MEAN = 3.5
_NC = 2
_NS = 16
_NW = _NC * _NS
_L = 16


@jax.jit
def _gmf_sc(uid, mid, uembt, membt, ubias, mbias, w_bcast):
    B = uid.shape[0]
    D = uembt.shape[0]
    BPW = B // _NW
    G = BPW // _L

    mesh = plsc.VectorSubcoreMesh(core_axis_name="c", subcore_axis_name="s")

    @functools.partial(
        pl.kernel,
        mesh=mesh,
        out_type=jax.ShapeDtypeStruct((B,), jnp.float32),
        compiler_params=pltpu.CompilerParams(
            needs_layout_passes=False, use_tc_tiling_on_sc=False),
        scratch_types=[
            pltpu.VMEM((BPW,), jnp.int32),
            pltpu.VMEM((BPW,), jnp.int32),
            pltpu.VMEM((D, BPW), jnp.float32),
            pltpu.VMEM((D, BPW), jnp.float32),
            pltpu.VMEM((BPW,), jnp.float32),
            pltpu.VMEM((BPW,), jnp.float32),
            pltpu.VMEM((D, _L), jnp.float32),
            pltpu.VMEM((BPW,), jnp.float32),
            pltpu.SemaphoreType.DMA,
            pltpu.SemaphoreType.DMA,
        ],
    )
    def body(uid_hbm, mid_hbm, uembt_hbm, membt_hbm, ub_hbm, mb_hbm, w_hbm,
             out_hbm, uidx, midx, utab, mtab, ubv, mbv, wv, ob, semi, sem):
        wid = lax.axis_index("s") * _NC + lax.axis_index("c")
        base = wid * BPW

        cpi = pltpu.async_copy(uid_hbm.at[pl.ds(base, BPW)], uidx, semi)
        cpj = pltpu.async_copy(mid_hbm.at[pl.ds(base, BPW)], midx, semi)
        pltpu.sync_copy(w_hbm, wv)
        cpi.wait()
        cpj.wait()

        copies = []
        for d in range(D):
            copies.append(pltpu.async_copy(
                uembt_hbm.at[d].at[uidx], utab.at[d], sem))
            copies.append(pltpu.async_copy(
                membt_hbm.at[d].at[midx], mtab.at[d], sem))
        copies.append(pltpu.async_copy(ub_hbm.at[uidx], ubv, sem))
        copies.append(pltpu.async_copy(mb_hbm.at[midx], mbv, sem))
        for cp in copies:
            cp.wait()

        def group_body(g, carry):
            sl = pl.ds(g * _L, _L)
            acc = jnp.zeros((_L,), jnp.float32)
            for d in range(D):
                acc = acc + utab[d, sl] * mtab[d, sl] * wv[d, :]
            ob[sl] = acc + ubv[sl] + mbv[sl] + _GLOBAL_MEAN
            return carry

        lax.fori_loop(0, G, group_body, 0)

        pltpu.sync_copy(ob, out_hbm.at[pl.ds(base, BPW)])

    return body(uid, mid, uembt, membt, ubias, mbias, w_bcast)


def kernel(user_ids, movie_ids, user_emb, movie_emb, user_bias, movie_bias,
           affine_w):
    uid = user_ids.astype(jnp.int32)
    mid = movie_ids.astype(jnp.int32)
    D = user_emb.shape[1]
    w_bcast = jnp.broadcast_to(affine_w.reshape(D, 1), (D, _L)).astype(
        jnp.float32)
    return _gmf_sc(uid, mid, user_emb.T, movie_emb.T, user_bias.reshape(-1),
                   movie_bias.reshape(-1), w_bcast)
